# Initial kernel scaffold; baseline (speedup 1.0000x reference)
import jax
import jax.numpy as jnp
from jax.experimental import pallas as pl


def kernel(xyz1, xyz2, points1, points2, W1, gamma1, beta1, W2, gamma2, beta2):
    raise NotImplementedError("write your pallas kernel here")


# speedup vs baseline: 15.1289x; 15.1289x over previous
import functools

import jax
import jax.numpy as jnp
from jax.experimental import pallas as pl


def _top3_body(n2, x1_ref, x2_ref, idx_ref, w_ref):
    x1 = x1_ref[0]
    x2 = x2_ref[0]
    x1sq = jnp.sum(x1 * x1, axis=0)
    x2sq = jnp.sum(x2 * x2, axis=0)
    dot = jax.lax.dot_general(x1, x2, (((0,), (0,)), ((), ())),
                              preferred_element_type=jnp.float32)
    d = x1sq[:, None] + x2sq[None, :] - 2.0 * dot
    iota = jax.lax.broadcasted_iota(jnp.int32, d.shape, 1)
    dists = []
    for k in range(3):
        m = jnp.min(d, axis=1, keepdims=True)
        sel = jnp.min(jnp.where(d == m, iota, n2), axis=1)
        dists.append(m[:, 0])
        idx_ref[0, k] = sel
        d = jnp.where(iota == sel[:, None], jnp.float32(1e30), d)
    dist = jnp.maximum(jnp.stack(dists, axis=0), 1e-10)
    invd = 1.0 / dist
    w_ref[0] = invd / jnp.sum(invd, axis=0, keepdims=True)


def _t2_body(p2_ref, w1a_ref, out_ref):
    out_ref[0] = jnp.dot(p2_ref[0], w1a_ref[...],
                         preferred_element_type=jnp.float32)


def _h1_body(n2, idx_ref, w_ref, t2_ref, p1_ref, w1b_ref,
             h1_ref, sum_ref, ssq_ref):
    b = pl.program_id(0)
    t = pl.program_id(1)
    idx = idx_ref[0]
    w = w_ref[0]
    tn = idx.shape[1]
    iota = jax.lax.broadcasted_iota(jnp.int32, (tn, n2), 1)
    s = jnp.where(idx[0][:, None] == iota, w[0][:, None], 0.0)
    s = s + jnp.where(idx[1][:, None] == iota, w[1][:, None], 0.0)
    s = s + jnp.where(idx[2][:, None] == iota, w[2][:, None], 0.0)
    h = jnp.dot(s, t2_ref[0], preferred_element_type=jnp.float32)
    h = h + jnp.dot(p1_ref[0], w1b_ref[...],
                    preferred_element_type=jnp.float32)
    h1_ref[0] = h

    @pl.when(jnp.logical_and(b == 0, t == 0))
    def _init():
        sum_ref[...] = jnp.zeros_like(sum_ref)
        ssq_ref[...] = jnp.zeros_like(ssq_ref)

    sum_ref[...] += jnp.sum(h, axis=0, keepdims=True)
    ssq_ref[...] += jnp.sum(h * h, axis=0, keepdims=True)


def _h2_body(h1_ref, sc_ref, bi_ref, w2_ref, h2_ref, sum_ref, ssq_ref):
    b = pl.program_id(0)
    t = pl.program_id(1)
    a = jnp.maximum(h1_ref[0] * sc_ref[...] + bi_ref[...], 0.0)
    h = jnp.dot(a, w2_ref[...], preferred_element_type=jnp.float32)
    h2_ref[0] = h

    @pl.when(jnp.logical_and(b == 0, t == 0))
    def _init():
        sum_ref[...] = jnp.zeros_like(sum_ref)
        ssq_ref[...] = jnp.zeros_like(ssq_ref)

    sum_ref[...] += jnp.sum(h, axis=0, keepdims=True)
    ssq_ref[...] += jnp.sum(h * h, axis=0, keepdims=True)


def _out_body(h2_ref, sc_ref, bi_ref, out_ref):
    out_ref[0] = jnp.maximum(h2_ref[0] * sc_ref[...] + bi_ref[...], 0.0)


def kernel(xyz1, xyz2, points1, points2, W1, gamma1, beta1, W2, gamma2,
           beta2):
    B, N1, _ = xyz1.shape
    N2 = xyz2.shape[1]
    C1 = points1.shape[2]
    C2 = points2.shape[2]
    H1 = W1.shape[1]
    H2 = W2.shape[1]
    TN = min(512, N1)
    NT = N1 // TN
    f32 = jnp.float32

    x1t = jnp.transpose(xyz1, (0, 2, 1))
    x2t = jnp.transpose(xyz2, (0, 2, 1))

    idx, wgt = pl.pallas_call(
        functools.partial(_top3_body, N2),
        grid=(B, NT),
        in_specs=[
            pl.BlockSpec((1, 3, TN), lambda b, t: (b, 0, t)),
            pl.BlockSpec((1, 3, N2), lambda b, t: (b, 0, 0)),
        ],
        out_specs=[
            pl.BlockSpec((1, 3, TN), lambda b, t: (b, 0, t)),
            pl.BlockSpec((1, 3, TN), lambda b, t: (b, 0, t)),
        ],
        out_shape=[
            jax.ShapeDtypeStruct((B, 3, N1), jnp.int32),
            jax.ShapeDtypeStruct((B, 3, N1), f32),
        ],
    )(x1t, x2t)

    W1a = W1[:C2]
    W1b = W1[C2:]

    t2 = pl.pallas_call(
        _t2_body,
        grid=(B,),
        in_specs=[
            pl.BlockSpec((1, N2, C2), lambda b: (b, 0, 0)),
            pl.BlockSpec((C2, H1), lambda b: (0, 0)),
        ],
        out_specs=pl.BlockSpec((1, N2, H1), lambda b: (b, 0, 0)),
        out_shape=jax.ShapeDtypeStruct((B, N2, H1), f32),
    )(points2, W1a)

    h1, s1, q1 = pl.pallas_call(
        functools.partial(_h1_body, N2),
        grid=(B, NT),
        in_specs=[
            pl.BlockSpec((1, 3, TN), lambda b, t: (b, 0, t)),
            pl.BlockSpec((1, 3, TN), lambda b, t: (b, 0, t)),
            pl.BlockSpec((1, N2, H1), lambda b, t: (b, 0, 0)),
            pl.BlockSpec((1, TN, C1), lambda b, t: (b, t, 0)),
            pl.BlockSpec((C1, H1), lambda b, t: (0, 0)),
        ],
        out_specs=[
            pl.BlockSpec((1, TN, H1), lambda b, t: (b, t, 0)),
            pl.BlockSpec((1, H1), lambda b, t: (0, 0)),
            pl.BlockSpec((1, H1), lambda b, t: (0, 0)),
        ],
        out_shape=[
            jax.ShapeDtypeStruct((B, N1, H1), f32),
            jax.ShapeDtypeStruct((1, H1), f32),
            jax.ShapeDtypeStruct((1, H1), f32),
        ],
    )(idx, wgt, t2, points1, W1b)

    n = B * N1
    mean1 = s1 / n
    var1 = q1 / n - mean1 * mean1
    scale1 = gamma1[None, :] / jnp.sqrt(var1 + 1e-3)
    bias1 = beta1[None, :] - mean1 * scale1

    h2, s2, q2 = pl.pallas_call(
        _h2_body,
        grid=(B, NT),
        in_specs=[
            pl.BlockSpec((1, TN, H1), lambda b, t: (b, t, 0)),
            pl.BlockSpec((1, H1), lambda b, t: (0, 0)),
            pl.BlockSpec((1, H1), lambda b, t: (0, 0)),
            pl.BlockSpec((H1, H2), lambda b, t: (0, 0)),
        ],
        out_specs=[
            pl.BlockSpec((1, TN, H2), lambda b, t: (b, t, 0)),
            pl.BlockSpec((1, H2), lambda b, t: (0, 0)),
            pl.BlockSpec((1, H2), lambda b, t: (0, 0)),
        ],
        out_shape=[
            jax.ShapeDtypeStruct((B, N1, H2), f32),
            jax.ShapeDtypeStruct((1, H2), f32),
            jax.ShapeDtypeStruct((1, H2), f32),
        ],
    )(h1, scale1, bias1, W2)

    mean2 = s2 / n
    var2 = q2 / n - mean2 * mean2
    scale2 = gamma2[None, :] / jnp.sqrt(var2 + 1e-3)
    bias2 = beta2[None, :] - mean2 * scale2

    out = pl.pallas_call(
        _out_body,
        grid=(B, NT),
        in_specs=[
            pl.BlockSpec((1, TN, H2), lambda b, t: (b, t, 0)),
            pl.BlockSpec((1, H2), lambda b, t: (0, 0)),
            pl.BlockSpec((1, H2), lambda b, t: (0, 0)),
        ],
        out_specs=pl.BlockSpec((1, TN, H2), lambda b, t: (b, t, 0)),
        out_shape=jax.ShapeDtypeStruct((B, N1, H2), f32),
    )(h2, scale2, bias2)

    return out


# speedup vs baseline: 17.0109x; 1.1244x over previous
import functools

import jax
import jax.numpy as jnp
from jax.experimental import pallas as pl


def _top3_body(n2, x1_ref, x2_ref, idx_ref, w_ref):
    x1 = x1_ref[0]
    x2 = x2_ref[0]
    x1sq = jnp.sum(x1 * x1, axis=0)
    x2sq = jnp.sum(x2 * x2, axis=0)
    dot = jax.lax.dot_general(x1, x2, (((0,), (0,)), ((), ())),
                              preferred_element_type=jnp.float32)
    d = x1sq[:, None] + x2sq[None, :] - 2.0 * dot
    tn = d.shape[0]
    rs = 64
    lane = jax.lax.broadcasted_iota(jnp.int32, (rs, 128), 1)
    big = jnp.float32(1e30)
    for r in range(tn // rs):
        dr = jax.lax.slice(d, (r * rs, 0), ((r + 1) * rs, n2))
        v1 = jnp.full((rs, 128), big)
        v2 = jnp.full((rs, 128), big)
        v3 = jnp.full((rs, 128), big)
        i1 = jnp.zeros((rs, 128), jnp.int32)
        i2 = jnp.zeros((rs, 128), jnp.int32)
        i3 = jnp.zeros((rs, 128), jnp.int32)
        for c in range(n2 // 128):
            cv = jax.lax.slice(dr, (0, c * 128), (rs, (c + 1) * 128))
            ci = lane + c * 128
            lt1 = cv < v1
            lt2 = cv < v2
            lt3 = cv < v3
            v3 = jnp.where(lt2, v2, jnp.where(lt3, cv, v3))
            i3 = jnp.where(lt2, i2, jnp.where(lt3, ci, i3))
            v2 = jnp.where(lt1, v1, jnp.where(lt2, cv, v2))
            i2 = jnp.where(lt1, i1, jnp.where(lt2, ci, i2))
            v1 = jnp.where(lt1, cv, v1)
            i1 = jnp.where(lt1, ci, i1)
        v = jnp.concatenate([v1, v2, v3], axis=1)
        iv = jnp.concatenate([i1, i2, i3], axis=1)
        dists = []
        for k in range(3):
            m = jnp.min(v, axis=1, keepdims=True)
            sel = jnp.min(jnp.where(v == m, iv, n2), axis=1)
            idx_ref[0, k, pl.ds(r * rs, rs)] = sel
            dists.append(m[:, 0])
            v = jnp.where(iv == sel[:, None], big, v)
        dist = jnp.maximum(jnp.stack(dists, axis=0), 1e-10)
        invd = 1.0 / dist
        w_ref[0, :, pl.ds(r * rs, rs)] = (
            invd / jnp.sum(invd, axis=0, keepdims=True))


def _t2_body(p2_ref, w1a_ref, out_ref):
    out_ref[0] = jnp.dot(p2_ref[0], w1a_ref[...],
                         preferred_element_type=jnp.float32)


def _h1_body(n2, idx_ref, w_ref, t2_ref, p1_ref, w1b_ref,
             h1_ref, sum_ref, ssq_ref):
    b = pl.program_id(0)
    t = pl.program_id(1)
    idx = idx_ref[0]
    w = w_ref[0]
    tn = idx.shape[1]
    iota = jax.lax.broadcasted_iota(jnp.int32, (tn, n2), 1)
    s = jnp.where(idx[0][:, None] == iota, w[0][:, None], 0.0)
    s = s + jnp.where(idx[1][:, None] == iota, w[1][:, None], 0.0)
    s = s + jnp.where(idx[2][:, None] == iota, w[2][:, None], 0.0)
    h = jnp.dot(s, t2_ref[0], preferred_element_type=jnp.float32)
    h = h + jnp.dot(p1_ref[0], w1b_ref[...],
                    preferred_element_type=jnp.float32)
    h1_ref[0] = h

    @pl.when(jnp.logical_and(b == 0, t == 0))
    def _init():
        sum_ref[...] = jnp.zeros_like(sum_ref)
        ssq_ref[...] = jnp.zeros_like(ssq_ref)

    sum_ref[...] += jnp.sum(h, axis=0, keepdims=True)
    ssq_ref[...] += jnp.sum(h * h, axis=0, keepdims=True)


def _h2_body(h1_ref, sc_ref, bi_ref, w2_ref, h2_ref, sum_ref, ssq_ref):
    b = pl.program_id(0)
    t = pl.program_id(1)
    a = jnp.maximum(h1_ref[0] * sc_ref[...] + bi_ref[...], 0.0)
    h = jnp.dot(a, w2_ref[...], preferred_element_type=jnp.float32)
    h2_ref[0] = h

    @pl.when(jnp.logical_and(b == 0, t == 0))
    def _init():
        sum_ref[...] = jnp.zeros_like(sum_ref)
        ssq_ref[...] = jnp.zeros_like(ssq_ref)

    sum_ref[...] += jnp.sum(h, axis=0, keepdims=True)
    ssq_ref[...] += jnp.sum(h * h, axis=0, keepdims=True)


def _out_body(h2_ref, sc_ref, bi_ref, out_ref):
    out_ref[0] = jnp.maximum(h2_ref[0] * sc_ref[...] + bi_ref[...], 0.0)


def kernel(xyz1, xyz2, points1, points2, W1, gamma1, beta1, W2, gamma2,
           beta2):
    B, N1, _ = xyz1.shape
    N2 = xyz2.shape[1]
    C1 = points1.shape[2]
    C2 = points2.shape[2]
    H1 = W1.shape[1]
    H2 = W2.shape[1]
    TN = min(512, N1)
    NT = N1 // TN
    f32 = jnp.float32

    x1t = jnp.transpose(xyz1, (0, 2, 1))
    x2t = jnp.transpose(xyz2, (0, 2, 1))

    idx, wgt = pl.pallas_call(
        functools.partial(_top3_body, N2),
        grid=(B, NT),
        in_specs=[
            pl.BlockSpec((1, 3, TN), lambda b, t: (b, 0, t)),
            pl.BlockSpec((1, 3, N2), lambda b, t: (b, 0, 0)),
        ],
        out_specs=[
            pl.BlockSpec((1, 3, TN), lambda b, t: (b, 0, t)),
            pl.BlockSpec((1, 3, TN), lambda b, t: (b, 0, t)),
        ],
        out_shape=[
            jax.ShapeDtypeStruct((B, 3, N1), jnp.int32),
            jax.ShapeDtypeStruct((B, 3, N1), f32),
        ],
    )(x1t, x2t)

    W1a = W1[:C2]
    W1b = W1[C2:]
    W2b = W2

    t2 = pl.pallas_call(
        _t2_body,
        grid=(B,),
        in_specs=[
            pl.BlockSpec((1, N2, C2), lambda b: (b, 0, 0)),
            pl.BlockSpec((C2, H1), lambda b: (0, 0)),
        ],
        out_specs=pl.BlockSpec((1, N2, H1), lambda b: (b, 0, 0)),
        out_shape=jax.ShapeDtypeStruct((B, N2, H1), f32),
    )(points2, W1a)

    h1, s1, q1 = pl.pallas_call(
        functools.partial(_h1_body, N2),
        grid=(B, NT),
        in_specs=[
            pl.BlockSpec((1, 3, TN), lambda b, t: (b, 0, t)),
            pl.BlockSpec((1, 3, TN), lambda b, t: (b, 0, t)),
            pl.BlockSpec((1, N2, H1), lambda b, t: (b, 0, 0)),
            pl.BlockSpec((1, TN, C1), lambda b, t: (b, t, 0)),
            pl.BlockSpec((C1, H1), lambda b, t: (0, 0)),
        ],
        out_specs=[
            pl.BlockSpec((1, TN, H1), lambda b, t: (b, t, 0)),
            pl.BlockSpec((1, H1), lambda b, t: (0, 0)),
            pl.BlockSpec((1, H1), lambda b, t: (0, 0)),
        ],
        out_shape=[
            jax.ShapeDtypeStruct((B, N1, H1), f32),
            jax.ShapeDtypeStruct((1, H1), f32),
            jax.ShapeDtypeStruct((1, H1), f32),
        ],
    )(idx, wgt, t2, points1, W1b)

    n = B * N1
    mean1 = s1 / n
    var1 = q1 / n - mean1 * mean1
    scale1 = gamma1[None, :] / jnp.sqrt(var1 + 1e-3)
    bias1 = beta1[None, :] - mean1 * scale1

    h2, s2, q2 = pl.pallas_call(
        _h2_body,
        grid=(B, NT),
        in_specs=[
            pl.BlockSpec((1, TN, H1), lambda b, t: (b, t, 0)),
            pl.BlockSpec((1, H1), lambda b, t: (0, 0)),
            pl.BlockSpec((1, H1), lambda b, t: (0, 0)),
            pl.BlockSpec((H1, H2), lambda b, t: (0, 0)),
        ],
        out_specs=[
            pl.BlockSpec((1, TN, H2), lambda b, t: (b, t, 0)),
            pl.BlockSpec((1, H2), lambda b, t: (0, 0)),
            pl.BlockSpec((1, H2), lambda b, t: (0, 0)),
        ],
        out_shape=[
            jax.ShapeDtypeStruct((B, N1, H2), f32),
            jax.ShapeDtypeStruct((1, H2), f32),
            jax.ShapeDtypeStruct((1, H2), f32),
        ],
    )(h1, scale1, bias1, W2b)

    mean2 = s2 / n
    var2 = q2 / n - mean2 * mean2
    scale2 = gamma2[None, :] / jnp.sqrt(var2 + 1e-3)
    bias2 = beta2[None, :] - mean2 * scale2

    out = pl.pallas_call(
        _out_body,
        grid=(B, NT),
        in_specs=[
            pl.BlockSpec((1, TN, H2), lambda b, t: (b, t, 0)),
            pl.BlockSpec((1, H2), lambda b, t: (0, 0)),
            pl.BlockSpec((1, H2), lambda b, t: (0, 0)),
        ],
        out_specs=pl.BlockSpec((1, TN, H2), lambda b, t: (b, t, 0)),
        out_shape=jax.ShapeDtypeStruct((B, N1, H2), f32),
    )(h2, scale2, bias2)

    return out
